# baseline (speedup 1.0000x reference)
import functools

import jax
import jax.numpy as jnp
from jax import lax
from jax.experimental import pallas as pl
from jax.experimental.pallas import tpu as pltpu
from jax.experimental.pallas import tpu_sc as plsc

RADIUS2 = 0.4 * 0.4
NS = 32
EPS = 1e-5
B = 2
N = 4096
M = 1024
C = 128
CIN = C + 3
NTILES = 32
QPT = (B * M) // NTILES
SLACK = 64
RPT = (B * M * NS) // NTILES
GCH = 128
ROWS = B * M * NS
BLK = 4096


def _mm_body(x_ref, w_ref, o_ref):
    o_ref[...] = jnp.dot(x_ref[...], w_ref[...],
                         preferred_element_type=jnp.float32)


def _layer1(x_all, w1t):
    return pl.pallas_call(
        _mm_body,
        out_shape=jax.ShapeDtypeStruct((B * N + B * M, C), jnp.float32),
    )(x_all, w1t)


def _bq_gather(xyz_flat, g_rows):
    mesh = plsc.VectorSubcoreMesh(core_axis_name="c", subcore_axis_name="s",
                                  num_cores=2, num_subcores=16)
    nch = RPT // GCH

    @functools.partial(
        pl.kernel,
        out_type=jax.ShapeDtypeStruct((ROWS, C), jnp.float32),
        mesh=mesh,
        compiler_params=pltpu.CompilerParams(needs_layout_passes=False),
        scratch_types=[
            pltpu.VMEM((N,), jnp.float32),
            pltpu.VMEM((N,), jnp.float32),
            pltpu.VMEM((N,), jnp.float32),
            pltpu.VMEM((QPT * SLACK,), jnp.int32),
            pltpu.VMEM((QPT * NS,), jnp.int32),
            pltpu.VMEM((2, GCH), jnp.int32),
            pltpu.VMEM((2, GCH, C), jnp.float32),
            pltpu.SemaphoreType.DMA,
            pltpu.SemaphoreType.DMA,
            pltpu.SemaphoreType.DMA,
            pltpu.SemaphoreType.DMA,
        ],
    )
    def bq(xyz_hbm, g_hbm, out_hbm, x_v, y_v, z_v, buf_v, idx_v, ich_v,
           ring_v, gs0, gs1, ws0, ws1):
        w = lax.axis_index("c") * 16 + lax.axis_index("s")
        b = w // (NTILES // B)
        pltpu.sync_copy(xyz_hbm.at[pl.ds((b * 3 + 0) * N, N)], x_v)
        pltpu.sync_copy(xyz_hbm.at[pl.ds((b * 3 + 1) * N, N)], y_v)
        pltpu.sync_copy(xyz_hbm.at[pl.ds((b * 3 + 2) * N, N)], z_v)
        it16 = lax.iota(jnp.int32, 16)

        PRE = 512

        def per_q(q, carry):
            gq = w * QPT + q
            m = gq - b * M
            qi = m // 32
            qj = m - qi * 32
            n = 128 * qi + 2 * qj
            nv = jnp.full((16,), n, jnp.int32)
            qx = plsc.load_gather(x_v, [nv])
            qy = plsc.load_gather(y_v, [nv])
            qz = plsc.load_gather(z_v, [nv])

            def scan32(base, cnt):
                px0 = x_v[pl.ds(base, 16)]
                py0 = y_v[pl.ds(base, 16)]
                pz0 = z_v[pl.ds(base, 16)]
                px1 = x_v[pl.ds(base + 16, 16)]
                py1 = y_v[pl.ds(base + 16, 16)]
                pz1 = z_v[pl.ds(base + 16, 16)]
                dx0 = px0 - qx
                dy0 = py0 - qy
                dz0 = pz0 - qz
                dx1 = px1 - qx
                dy1 = py1 - qy
                dz1 = pz1 - qz
                d20 = dx0 * dx0 + dy0 * dy0 + dz0 * dz0
                d21 = dx1 * dx1 + dy1 * dy1 + dz1 * dz1
                m0 = d20 < RADIUS2
                m1 = d21 < RADIUS2
                m0i = m0.astype(jnp.int32)
                m1i = m1.astype(jnp.int32)
                pc0 = plsc.all_reduce_population_count(m0)
                pc1 = plsc.all_reduce_population_count(m1)
                s0 = cnt + plsc.cumsum(m0i) - m0i
                s1 = cnt + pc0 + plsc.cumsum(m1i) - m1i
                wm0 = m0 & (s0 < SLACK)
                wm1 = m1 & (s1 < SLACK)
                plsc.store_scatter(buf_v, [q * SLACK + s0],
                                   base + it16 + b * N, mask=wm0)
                plsc.store_scatter(buf_v, [q * SLACK + s1],
                                   base + 16 + it16 + b * N, mask=wm1)
                return cnt + pc0 + pc1

            cnt = jnp.zeros((16,), jnp.int32)
            for chs in range(PRE // 32):
                cnt = scan32(chs * 32, cnt)

            def cond(st):
                ch, c = st
                return (ch < N // 64) & (jnp.max(c) < NS)

            def step(st):
                ch, c = st
                base = ch * 64
                c = scan32(base, c)
                c = scan32(base + 32, c)
                return ch + 1, c

            _, cntf = lax.while_loop(cond, step, (jnp.int32(PRE // 64), cnt))
            firstv = plsc.load_gather(
                buf_v, [jnp.full((16,), q * SLACK, jnp.int32)])
            fill = jnp.where(cntf > 0, firstv,
                             jnp.full((16,), b * N, jnp.int32))
            for h in range(NS // 16):
                sl = it16 + h * 16
                cur = buf_v[pl.ds(q * SLACK + h * 16, 16)]
                idx_v[pl.ds(q * NS + h * 16, 16)] = jnp.where(
                    sl < cntf, cur, fill)
            return carry

        lax.fori_loop(0, QPT, per_q, 0)

        gsems = [gs0, gs1]
        wsems = [ws0, ws1]

        def prep_idx(kk):
            s = kk % 2
            for v in range(GCH // 16):
                ich_v[s, pl.ds(v * 16, 16)] = idx_v[pl.ds(kk * GCH + v * 16, 16)]

        def fire_gather(kk):
            s = kk % 2
            return pltpu.async_copy(g_hbm.at[ich_v.at[s]], ring_v.at[s],
                                    gsems[s])

        def fire_write(kk):
            s = kk % 2
            base = w * RPT + kk * GCH
            return pltpu.async_copy(ring_v.at[s],
                                    out_hbm.at[pl.ds(base, GCH)], wsems[s])

        gcps = {}
        wcps = [None, None]
        prep_idx(0)
        gcps[0] = fire_gather(0)
        for kk in range(nch):
            nxt = kk + 1
            if nxt < nch:
                if wcps[nxt % 2] is not None:
                    wcps[nxt % 2].wait()
                prep_idx(nxt)
                gcps[nxt] = fire_gather(nxt)
            gcps[kk].wait()
            wcps[kk % 2] = fire_write(kk)
        wcps[(nch - 1) % 2].wait()
        if wcps[nch % 2] is not None:
            wcps[nch % 2].wait()

    return bq(xyz_flat, g_rows)


def _stats_body(gg_ref, qr_ref, sum_ref, sq_ref):
    z3 = gg_ref[...].reshape(BLK // NS, NS, C) - qr_ref[...][:, None, :]
    z = z3.reshape(BLK, C)

    @pl.when(pl.program_id(0) == 0)
    def _():
        sum_ref[...] = jnp.zeros_like(sum_ref)
        sq_ref[...] = jnp.zeros_like(sq_ref)

    sum_ref[...] += jnp.sum(z, axis=0)[None]
    sq_ref[...] += jnp.sum(z * z, axis=0)[None]


def _stats1(gg, q_rows):
    grid = ROWS // BLK
    return pl.pallas_call(
        _stats_body,
        grid=(grid,),
        in_specs=[
            pl.BlockSpec((BLK, C), lambda i: (i, 0)),
            pl.BlockSpec((BLK // NS, C), lambda i: (i, 0)),
        ],
        out_specs=[
            pl.BlockSpec((1, C), lambda i: (0, 0)),
            pl.BlockSpec((1, C), lambda i: (0, 0)),
        ],
        out_shape=[
            jax.ShapeDtypeStruct((1, C), jnp.float32),
            jax.ShapeDtypeStruct((1, C), jnp.float32),
        ],
    )(gg, q_rows)


def _main_body(gg_ref, qr_ref, sum1_ref, sq1_ref, g1_ref, b1_ref, w2t_ref,
               maxz_ref, sum2_ref, sq2_ref):
    cnt = jnp.float32(ROWS)
    mean = sum1_ref[...] / cnt
    var = sq1_ref[...] / cnt - mean * mean
    s1 = g1_ref[...] * lax.rsqrt(var + EPS)
    t1 = b1_ref[...] - mean * s1
    z3 = gg_ref[...].reshape(BLK // NS, NS, C) - qr_ref[...][:, None, :]
    z1 = z3.reshape(BLK, C)
    a1 = jnp.maximum(z1 * s1 + t1, 0.0)
    z2 = jnp.dot(a1, w2t_ref[...], preferred_element_type=jnp.float32)

    @pl.when(pl.program_id(0) == 0)
    def _():
        sum2_ref[...] = jnp.zeros_like(sum2_ref)
        sq2_ref[...] = jnp.zeros_like(sq2_ref)

    sum2_ref[...] += jnp.sum(z2, axis=0)[None]
    sq2_ref[...] += jnp.sum(z2 * z2, axis=0)[None]
    maxz_ref[...] = jnp.max(z2.reshape(BLK // NS, NS, 2 * C), axis=1)


def _main(gg, q_rows, sum1, sq1, g1, b1, w2t):
    grid = ROWS // BLK
    return pl.pallas_call(
        _main_body,
        grid=(grid,),
        in_specs=[
            pl.BlockSpec((BLK, C), lambda i: (i, 0)),
            pl.BlockSpec((BLK // NS, C), lambda i: (i, 0)),
            pl.BlockSpec((1, C), lambda i: (0, 0)),
            pl.BlockSpec((1, C), lambda i: (0, 0)),
            pl.BlockSpec((1, C), lambda i: (0, 0)),
            pl.BlockSpec((1, C), lambda i: (0, 0)),
            pl.BlockSpec((C, 2 * C), lambda i: (0, 0)),
        ],
        out_specs=[
            pl.BlockSpec((BLK // NS, 2 * C), lambda i: (i, 0)),
            pl.BlockSpec((1, 2 * C), lambda i: (0, 0)),
            pl.BlockSpec((1, 2 * C), lambda i: (0, 0)),
        ],
        out_shape=[
            jax.ShapeDtypeStruct((B * M, 2 * C), jnp.float32),
            jax.ShapeDtypeStruct((1, 2 * C), jnp.float32),
            jax.ShapeDtypeStruct((1, 2 * C), jnp.float32),
        ],
    )(gg, q_rows, sum1, sq1, g1, b1, w2t)


def _final_body(mz_ref, sum2_ref, sq2_ref, g2_ref, b2_ref, o_ref):
    cnt = jnp.float32(ROWS)
    mean = sum2_ref[...] / cnt
    var = sq2_ref[...] / cnt - mean * mean
    s2 = g2_ref[...] * lax.rsqrt(var + EPS)
    t2 = b2_ref[...] - mean * s2
    o_ref[...] = jnp.maximum(mz_ref[...] * s2 + t2, 0.0)


def _final(maxz, sum2, sq2, g2, b2):
    return pl.pallas_call(
        _final_body,
        out_shape=jax.ShapeDtypeStruct((B * M, 2 * C), jnp.float32),
    )(maxz, sum2, sq2, g2, b2)


def kernel(xyz, features, W1, g1, b1, W2, g2, b2):
    new_xyz_img = xyz[:, :, ::2, ::2]
    xyz_flat = xyz.reshape(B, 3, N)
    xyz_pts = jnp.transpose(xyz_flat, (0, 2, 1))
    feat_t = jnp.transpose(features.reshape(B, C, N), (0, 2, 1))
    x1 = jnp.concatenate([xyz_pts, feat_t], axis=-1).reshape(B * N, CIN)
    new_pts = jnp.transpose(new_xyz_img.reshape(B, 3, M), (0, 2, 1))
    x2 = jnp.concatenate(
        [new_pts, jnp.zeros((B, M, C), jnp.float32)], axis=-1
    ).reshape(B * M, CIN)
    x_all = jnp.concatenate([x1, x2], axis=0)

    g_all = _layer1(x_all, W1.T)
    g_rows = g_all[: B * N]
    q_rows = g_all[B * N:]

    gg = _bq_gather(xyz_flat.reshape(B * 3 * N), g_rows)

    sum1, sq1 = _stats1(gg, q_rows)
    maxz, sum2, sq2 = _main(gg, q_rows, sum1, sq1,
                            g1.reshape(1, C), b1.reshape(1, C), W2.T)
    out = _final(maxz, sum2, sq2, g2.reshape(1, 2 * C), b2.reshape(1, 2 * C))
    out_img = out.reshape(B, M, 2 * C).transpose(0, 2, 1).reshape(B, 2 * C, 32, 32)
    return (new_xyz_img, out_img)


# speedup vs baseline: 1.0482x; 1.0482x over previous
import functools

import jax
import jax.numpy as jnp
from jax import lax
from jax.experimental import pallas as pl
from jax.experimental.pallas import tpu as pltpu
from jax.experimental.pallas import tpu_sc as plsc

RADIUS2 = 0.4 * 0.4
NS = 32
EPS = 1e-5
B = 2
N = 4096
M = 1024
C = 128
CIN = C + 3
NTILES = 32
QPT = (B * M) // NTILES
SLACK = 64
RPT = (B * M * NS) // NTILES
GCH = 128
ROWS = B * M * NS
BLK = 4096


def _mm_body(x_ref, w_ref, o_ref):
    o_ref[...] = jnp.dot(x_ref[...], w_ref[...],
                         preferred_element_type=jnp.float32)


def _layer1(x_all, w1t):
    return pl.pallas_call(
        _mm_body,
        out_shape=jax.ShapeDtypeStruct((B * N + B * M, C), jnp.float32),
    )(x_all, w1t)


def _bq_gather(xyz_flat, g_rows):
    mesh = plsc.VectorSubcoreMesh(core_axis_name="c", subcore_axis_name="s",
                                  num_cores=2, num_subcores=16)
    nch = RPT // GCH

    @functools.partial(
        pl.kernel,
        out_type=jax.ShapeDtypeStruct((ROWS, C), jnp.float32),
        mesh=mesh,
        compiler_params=pltpu.CompilerParams(needs_layout_passes=False),
        scratch_types=[
            pltpu.VMEM((N,), jnp.float32),
            pltpu.VMEM((N,), jnp.float32),
            pltpu.VMEM((N,), jnp.float32),
            pltpu.VMEM((QPT * SLACK,), jnp.int32),
            pltpu.VMEM((QPT * NS,), jnp.int32),
            pltpu.VMEM((2, GCH), jnp.int32),
            pltpu.VMEM((2, GCH, C), jnp.float32),
            pltpu.SemaphoreType.DMA,
            pltpu.SemaphoreType.DMA,
            pltpu.SemaphoreType.DMA,
            pltpu.SemaphoreType.DMA,
        ],
    )
    def bq(xyz_hbm, g_hbm, out_hbm, x_v, y_v, z_v, buf_v, idx_v, ich_v,
           ring_v, gs0, gs1, ws0, ws1):
        w = lax.axis_index("c") * 16 + lax.axis_index("s")
        b = w // (NTILES // B)
        pltpu.sync_copy(xyz_hbm.at[pl.ds((b * 3 + 0) * N, N)], x_v)
        pltpu.sync_copy(xyz_hbm.at[pl.ds((b * 3 + 1) * N, N)], y_v)
        pltpu.sync_copy(xyz_hbm.at[pl.ds((b * 3 + 2) * N, N)], z_v)
        it16 = lax.iota(jnp.int32, 16)

        PRE = 256

        def per_q(q, carry):
            gq = w * QPT + q
            m = gq - b * M
            qi = m // 32
            qj = m - qi * 32
            n = 128 * qi + 2 * qj
            nv = jnp.full((16,), n, jnp.int32)
            qx = plsc.load_gather(x_v, [nv])
            qy = plsc.load_gather(y_v, [nv])
            qz = plsc.load_gather(z_v, [nv])

            def scan32(base, cnt):
                px0 = x_v[pl.ds(base, 16)]
                py0 = y_v[pl.ds(base, 16)]
                pz0 = z_v[pl.ds(base, 16)]
                px1 = x_v[pl.ds(base + 16, 16)]
                py1 = y_v[pl.ds(base + 16, 16)]
                pz1 = z_v[pl.ds(base + 16, 16)]
                dx0 = px0 - qx
                dy0 = py0 - qy
                dz0 = pz0 - qz
                dx1 = px1 - qx
                dy1 = py1 - qy
                dz1 = pz1 - qz
                d20 = dx0 * dx0 + dy0 * dy0 + dz0 * dz0
                d21 = dx1 * dx1 + dy1 * dy1 + dz1 * dz1
                m0 = d20 < RADIUS2
                m1 = d21 < RADIUS2
                m0i = m0.astype(jnp.int32)
                m1i = m1.astype(jnp.int32)
                pc0 = plsc.all_reduce_population_count(m0)
                pc1 = plsc.all_reduce_population_count(m1)
                s0 = cnt + plsc.cumsum(m0i) - m0i
                s1 = cnt + pc0 + plsc.cumsum(m1i) - m1i
                wm0 = m0 & (s0 < SLACK)
                wm1 = m1 & (s1 < SLACK)
                plsc.store_scatter(buf_v, [q * SLACK + s0],
                                   base + it16 + b * N, mask=wm0)
                plsc.store_scatter(buf_v, [q * SLACK + s1],
                                   base + 16 + it16 + b * N, mask=wm1)
                return cnt + pc0 + pc1

            cnt = jnp.zeros((16,), jnp.int32)
            for chs in range(PRE // 32):
                cnt = scan32(chs * 32, cnt)

            def cond(st):
                ch, c = st
                return (ch < N // 64) & (jnp.max(c) < NS)

            def step(st):
                ch, c = st
                base = ch * 64
                c = scan32(base, c)
                c = scan32(base + 32, c)
                return ch + 1, c

            _, cntf = lax.while_loop(cond, step, (jnp.int32(PRE // 64), cnt))
            firstv = plsc.load_gather(
                buf_v, [jnp.full((16,), q * SLACK, jnp.int32)])
            fill = jnp.where(cntf > 0, firstv,
                             jnp.full((16,), b * N, jnp.int32))
            for h in range(NS // 16):
                sl = it16 + h * 16
                cur = buf_v[pl.ds(q * SLACK + h * 16, 16)]
                idx_v[pl.ds(q * NS + h * 16, 16)] = jnp.where(
                    sl < cntf, cur, fill)
            return carry

        lax.fori_loop(0, QPT, per_q, 0)

        gsems = [gs0, gs1]
        wsems = [ws0, ws1]

        def prep_idx(kk):
            s = kk % 2
            for v in range(GCH // 16):
                ich_v[s, pl.ds(v * 16, 16)] = idx_v[pl.ds(kk * GCH + v * 16, 16)]

        def fire_gather(kk):
            s = kk % 2
            return pltpu.async_copy(g_hbm.at[ich_v.at[s]], ring_v.at[s],
                                    gsems[s])

        def fire_write(kk):
            s = kk % 2
            base = w * RPT + kk * GCH
            return pltpu.async_copy(ring_v.at[s],
                                    out_hbm.at[pl.ds(base, GCH)], wsems[s])

        gcps = {}
        wcps = [None, None]
        prep_idx(0)
        gcps[0] = fire_gather(0)
        for kk in range(nch):
            nxt = kk + 1
            if nxt < nch:
                if wcps[nxt % 2] is not None:
                    wcps[nxt % 2].wait()
                prep_idx(nxt)
                gcps[nxt] = fire_gather(nxt)
            gcps[kk].wait()
            wcps[kk % 2] = fire_write(kk)
        wcps[(nch - 1) % 2].wait()
        if wcps[nch % 2] is not None:
            wcps[nch % 2].wait()

    return bq(xyz_flat, g_rows)


def _stats_body(gg_ref, qr_ref, sum_ref, sq_ref):
    z3 = gg_ref[...].reshape(BLK // NS, NS, C) - qr_ref[...][:, None, :]
    z = z3.reshape(BLK, C)

    @pl.when(pl.program_id(0) == 0)
    def _():
        sum_ref[...] = jnp.zeros_like(sum_ref)
        sq_ref[...] = jnp.zeros_like(sq_ref)

    sum_ref[...] += jnp.sum(z, axis=0)[None]
    sq_ref[...] += jnp.sum(z * z, axis=0)[None]


def _stats1(gg, q_rows):
    grid = ROWS // BLK
    return pl.pallas_call(
        _stats_body,
        grid=(grid,),
        in_specs=[
            pl.BlockSpec((BLK, C), lambda i: (i, 0)),
            pl.BlockSpec((BLK // NS, C), lambda i: (i, 0)),
        ],
        out_specs=[
            pl.BlockSpec((1, C), lambda i: (0, 0)),
            pl.BlockSpec((1, C), lambda i: (0, 0)),
        ],
        out_shape=[
            jax.ShapeDtypeStruct((1, C), jnp.float32),
            jax.ShapeDtypeStruct((1, C), jnp.float32),
        ],
    )(gg, q_rows)


def _main_body(gg_ref, qr_ref, sum1_ref, sq1_ref, g1_ref, b1_ref, w2t_ref,
               maxz_ref, sum2_ref, sq2_ref):
    cnt = jnp.float32(ROWS)
    mean = sum1_ref[...] / cnt
    var = sq1_ref[...] / cnt - mean * mean
    s1 = g1_ref[...] * lax.rsqrt(var + EPS)
    t1 = b1_ref[...] - mean * s1
    z3 = gg_ref[...].reshape(BLK // NS, NS, C) - qr_ref[...][:, None, :]
    z1 = z3.reshape(BLK, C)
    a1 = jnp.maximum(z1 * s1 + t1, 0.0)
    z2 = jnp.dot(a1, w2t_ref[...], preferred_element_type=jnp.float32)

    @pl.when(pl.program_id(0) == 0)
    def _():
        sum2_ref[...] = jnp.zeros_like(sum2_ref)
        sq2_ref[...] = jnp.zeros_like(sq2_ref)

    sum2_ref[...] += jnp.sum(z2, axis=0)[None]
    sq2_ref[...] += jnp.sum(z2 * z2, axis=0)[None]
    maxz_ref[...] = jnp.max(z2.reshape(BLK // NS, NS, 2 * C), axis=1)


def _main(gg, q_rows, sum1, sq1, g1, b1, w2t):
    grid = ROWS // BLK
    return pl.pallas_call(
        _main_body,
        grid=(grid,),
        in_specs=[
            pl.BlockSpec((BLK, C), lambda i: (i, 0)),
            pl.BlockSpec((BLK // NS, C), lambda i: (i, 0)),
            pl.BlockSpec((1, C), lambda i: (0, 0)),
            pl.BlockSpec((1, C), lambda i: (0, 0)),
            pl.BlockSpec((1, C), lambda i: (0, 0)),
            pl.BlockSpec((1, C), lambda i: (0, 0)),
            pl.BlockSpec((C, 2 * C), lambda i: (0, 0)),
        ],
        out_specs=[
            pl.BlockSpec((BLK // NS, 2 * C), lambda i: (i, 0)),
            pl.BlockSpec((1, 2 * C), lambda i: (0, 0)),
            pl.BlockSpec((1, 2 * C), lambda i: (0, 0)),
        ],
        out_shape=[
            jax.ShapeDtypeStruct((B * M, 2 * C), jnp.float32),
            jax.ShapeDtypeStruct((1, 2 * C), jnp.float32),
            jax.ShapeDtypeStruct((1, 2 * C), jnp.float32),
        ],
    )(gg, q_rows, sum1, sq1, g1, b1, w2t)


def _final_body(mz_ref, sum2_ref, sq2_ref, g2_ref, b2_ref, o_ref):
    cnt = jnp.float32(ROWS)
    mean = sum2_ref[...] / cnt
    var = sq2_ref[...] / cnt - mean * mean
    s2 = g2_ref[...] * lax.rsqrt(var + EPS)
    t2 = b2_ref[...] - mean * s2
    o_ref[...] = jnp.maximum(mz_ref[...] * s2 + t2, 0.0)


def _final(maxz, sum2, sq2, g2, b2):
    return pl.pallas_call(
        _final_body,
        out_shape=jax.ShapeDtypeStruct((B * M, 2 * C), jnp.float32),
    )(maxz, sum2, sq2, g2, b2)


def kernel(xyz, features, W1, g1, b1, W2, g2, b2):
    new_xyz_img = xyz[:, :, ::2, ::2]
    xyz_flat = xyz.reshape(B, 3, N)
    xyz_pts = jnp.transpose(xyz_flat, (0, 2, 1))
    feat_t = jnp.transpose(features.reshape(B, C, N), (0, 2, 1))
    x1 = jnp.concatenate([xyz_pts, feat_t], axis=-1).reshape(B * N, CIN)
    new_pts = jnp.transpose(new_xyz_img.reshape(B, 3, M), (0, 2, 1))
    x2 = jnp.concatenate(
        [new_pts, jnp.zeros((B, M, C), jnp.float32)], axis=-1
    ).reshape(B * M, CIN)
    x_all = jnp.concatenate([x1, x2], axis=0)

    g_all = _layer1(x_all, W1.T)
    g_rows = g_all[: B * N]
    q_rows = g_all[B * N:]

    gg = _bq_gather(xyz_flat.reshape(B * 3 * N), g_rows)

    sum1, sq1 = _stats1(gg, q_rows)
    maxz, sum2, sq2 = _main(gg, q_rows, sum1, sq1,
                            g1.reshape(1, C), b1.reshape(1, C), W2.T)
    out = _final(maxz, sum2, sq2, g2.reshape(1, 2 * C), b2.reshape(1, 2 * C))
    out_img = out.reshape(B, M, 2 * C).transpose(0, 2, 1).reshape(B, 2 * C, 32, 32)
    return (new_xyz_img, out_img)
